# baseline (speedup 1.0000x reference)
import functools

import jax
import jax.numpy as jnp
from jax import lax
from jax.experimental import pallas as pl
from jax.experimental.pallas import tpu as pltpu
from jax.experimental.pallas import tpu_sc as plsc

N = 10000
K = 16
D = 128
B = 4
OUT = 2

NC, NS, L = 2, 16, 16
NW = NC * NS
ROWW = B * D
NPAD = 10240
PER_W = NPAD // NW
CHUNK = 4
N_CHUNKS = PER_W // CHUNK
NROWS = NPAD * B


def _gather_agg_body(x_hbm, adj_hbm, w_hbm, out_hbm, idx_v, rows_v, acc_v, w_v, sem):
    cid = lax.axis_index("c")
    sid = lax.axis_index("s")
    wid = sid * NC + cid
    base = wid * PER_W
    pltpu.sync_copy(w_hbm, w_v)

    def chunk_body(t, carry):
        n0 = base + t * CHUNK
        pltpu.sync_copy(adj_hbm.at[pl.ds(n0 * K, CHUNK * K)], idx_v)
        pltpu.async_copy(x_hbm.at[idx_v], rows_v, sem).wait()
        for j in range(CHUNK):
            for v in range(ROWW // L):
                sl = pl.ds(v * L, L)
                acc = rows_v[j * K, sl] * w_v[0, sl]
                for k in range(1, K):
                    acc = acc + rows_v[j * K + k, sl] * w_v[k, sl]
                acc_v[j, sl] = acc
        pltpu.sync_copy(acc_v, out_hbm.at[pl.ds(n0, CHUNK)])
        return carry

    lax.fori_loop(0, N_CHUNKS, chunk_body, 0)


@functools.partial(jax.jit, static_argnames=())
def _gather_agg(x_rows, adj_flat, w_bcast):
    mesh = plsc.VectorSubcoreMesh(core_axis_name="c", subcore_axis_name="s")
    return pl.kernel(
        _gather_agg_body,
        out_type=jax.ShapeDtypeStruct((NPAD, ROWW), jnp.float32),
        mesh=mesh,
        scratch_types=[
            pltpu.VMEM((CHUNK * K,), jnp.int32),
            pltpu.VMEM((CHUNK * K, ROWW), jnp.float32),
            pltpu.VMEM((CHUNK, ROWW), jnp.float32),
            pltpu.VMEM((K, ROWW), jnp.float32),
            pltpu.SemaphoreType.DMA,
        ],
    )(x_rows, adj_flat, w_bcast)


def _init_body(lat_ref, win_ref, bin_ref, pos_ref, out_ref):
    x0 = jnp.dot(lat_ref[...], win_ref[...],
                 preferred_element_type=jnp.float32) + bin_ref[...]
    out_ref[...] = pos_ref[...][:, None, :] + x0[None, :, :]


def _init_x(latent, W_in, b_in, pos_pad):
    tn = 1024
    return pl.pallas_call(
        _init_body,
        grid=(NPAD // tn,),
        in_specs=[
            pl.BlockSpec((B, W_in.shape[0]), lambda i: (0, 0)),
            pl.BlockSpec((W_in.shape[0], D), lambda i: (0, 0)),
            pl.BlockSpec((1, D), lambda i: (0, 0)),
            pl.BlockSpec((tn, D), lambda i: (i, 0)),
        ],
        out_specs=pl.BlockSpec((tn, B, D), lambda i: (i, 0, 0)),
        out_shape=jax.ShapeDtypeStruct((NPAD, B, D), jnp.float32),
    )(latent, W_in, b_in.reshape(1, D), pos_pad)


def _ln(x, g, b):
    m = jnp.mean(x, axis=-1, keepdims=True)
    v = jnp.mean((x - m) ** 2, axis=-1, keepdims=True)
    return (x - m) * lax.rsqrt(v + 1e-5) * g + b


def _block_body(x_ref, agg_ref, g_ref, b_ref, w1_ref, b1_ref, w2_ref, b2_ref,
                out_ref):
    h = _ln(agg_ref[...], g_ref[...], b_ref[...])
    u = jax.nn.gelu(jnp.dot(h, w1_ref[...], preferred_element_type=jnp.float32)
                    + b1_ref[...])
    y = jnp.dot(u, w2_ref[...], preferred_element_type=jnp.float32) + b2_ref[...]
    out_ref[...] = x_ref[...] + y


def _mlp_block(x2d, agg2d, g, b, W1, b1, W2, b2):
    r = 2048
    h4 = 4 * D
    return pl.pallas_call(
        _block_body,
        grid=(NROWS // r,),
        in_specs=[
            pl.BlockSpec((r, D), lambda i: (i, 0)),
            pl.BlockSpec((r, D), lambda i: (i, 0)),
            pl.BlockSpec((1, D), lambda i: (0, 0)),
            pl.BlockSpec((1, D), lambda i: (0, 0)),
            pl.BlockSpec((D, h4), lambda i: (0, 0)),
            pl.BlockSpec((1, h4), lambda i: (0, 0)),
            pl.BlockSpec((h4, D), lambda i: (0, 0)),
            pl.BlockSpec((1, D), lambda i: (0, 0)),
        ],
        out_specs=pl.BlockSpec((r, D), lambda i: (i, 0)),
        out_shape=jax.ShapeDtypeStruct((NROWS, D), jnp.float32),
    )(x2d, agg2d, g.reshape(1, D), b.reshape(1, D), W1, b1.reshape(1, h4),
      W2, b2.reshape(1, D))


def _head_body(x_ref, g_ref, b_ref, wh_ref, bh_ref, out_ref):
    h = _ln(x_ref[...], g_ref[...], b_ref[...])
    out_ref[...] = (jnp.dot(h, wh_ref[...], preferred_element_type=jnp.float32)
                    + bh_ref[...])


def _head(x2d, g, b, W_head, b_head):
    r = 2048
    return pl.pallas_call(
        _head_body,
        grid=(NROWS // r,),
        in_specs=[
            pl.BlockSpec((r, D), lambda i: (i, 0)),
            pl.BlockSpec((1, D), lambda i: (0, 0)),
            pl.BlockSpec((1, D), lambda i: (0, 0)),
            pl.BlockSpec((D, OUT), lambda i: (0, 0)),
            pl.BlockSpec((1, OUT), lambda i: (0, 0)),
        ],
        out_specs=pl.BlockSpec((r, OUT), lambda i: (i, 0)),
        out_shape=jax.ShapeDtypeStruct((NROWS, OUT), jnp.float32),
    )(x2d, g.reshape(1, D), b.reshape(1, D), W_head, b_head.reshape(1, OUT))


def kernel(latent_token, adj, W_in, b_in, pos_embed, w_nb, ln1_g, ln1_b,
           W1, b1, W2, b2, lnh_g, lnh_b, W_head, b_head):
    depth = w_nb.shape[0]
    pos_pad = jnp.zeros((NPAD, D), jnp.float32).at[:N].set(pos_embed[0])
    adj_flat = jnp.zeros((NPAD, K), jnp.int32).at[:N].set(
        adj.astype(jnp.int32)).reshape(NPAD * K)
    x = _init_x(latent_token, W_in, b_in, pos_pad)
    x = x.reshape(NPAD, ROWW)
    for i in range(depth):
        w_rows = jnp.tile(w_nb[i], (1, B)).astype(jnp.float32)
        agg = _gather_agg(x, adj_flat, w_rows)
        x = _mlp_block(x.reshape(NROWS, D), agg.reshape(NROWS, D),
                       ln1_g[i], ln1_b[i], W1[i], b1[i], W2[i], b2[i])
        x = x.reshape(NPAD, ROWW)
    y = _head(x.reshape(NROWS, D), lnh_g, lnh_b, W_head, b_head)
    out = y.reshape(NPAD, B, OUT)[:N]
    return jnp.transpose(out, (1, 2, 0))


# speedup vs baseline: 3.1857x; 3.1857x over previous
import functools

import jax
import jax.numpy as jnp
from jax import lax
from jax.experimental import pallas as pl
from jax.experimental.pallas import tpu as pltpu
from jax.experimental.pallas import tpu_sc as plsc

N = 10000
K = 16
D = 128
B = 4
OUT = 2

NC, NS, L = 2, 16, 16
NW = NC * NS
ROWW = B * D
NPAD = 10240
PER_W = NPAD // NW
CHUNK = 4
N_CHUNKS = PER_W // CHUNK
NROWS = NPAD * B


CK = CHUNK * K
NH = N_CHUNKS // 2


def _gather_agg_body(x_hbm, adj_hbm, w_hbm, out_hbm, adj_v, rows_a, rows_b,
                     acc_a, acc_b, w_v, sem_a, sem_b, sem_oa, sem_ob):
    cid = lax.axis_index("c")
    sid = lax.axis_index("s")
    wid = sid * NC + cid
    base = wid * PER_W
    pltpu.sync_copy(w_hbm, w_v)
    pltpu.sync_copy(adj_hbm.at[wid], adj_v)

    def gather(c, buf, sem):
        return pltpu.async_copy(x_hbm.at[adj_v.at[c]], buf, sem)

    def wait_gather(buf, sem):
        pltpu.make_async_copy(x_hbm.at[pl.ds(0, CK)], buf, sem).wait()

    def wait_scatter(acc, sem):
        pltpu.make_async_copy(acc, out_hbm.at[pl.ds(0, CHUNK)], sem).wait()

    def compute(c, buf, acc):
        def vbody(v, carry):
            sl = pl.ds(v * L, L)
            wv = [w_v[k, sl] for k in range(K)]
            for j in range(CHUNK):
                t = buf[j * K, sl] * wv[0]
                for k in range(1, K):
                    t = t + buf[j * K + k, sl] * wv[k]
                acc[j, sl] = t
            return carry

        lax.fori_loop(0, ROWW // L, vbody, 0)
        return pltpu.async_copy(
            acc, out_hbm.at[pl.ds(base + c * CHUNK, CHUNK)],
            sem_oa if acc is acc_a else sem_ob)

    gather(0, rows_a, sem_a)

    def body(t, carry):
        c0 = 2 * t
        c1 = 2 * t + 1
        gather(c1, rows_b, sem_b)
        wait_gather(rows_a, sem_a)

        @pl.when(t > 0)
        def _():
            wait_scatter(acc_a, sem_oa)
        compute(c0, rows_a, acc_a)

        @pl.when(t < NH - 1)
        def _():
            gather(c0 + 2, rows_a, sem_a)
        wait_gather(rows_b, sem_b)

        @pl.when(t > 0)
        def _():
            wait_scatter(acc_b, sem_ob)
        compute(c1, rows_b, acc_b)
        return carry

    lax.fori_loop(0, NH, body, 0)
    wait_scatter(acc_a, sem_oa)
    wait_scatter(acc_b, sem_ob)


@functools.partial(jax.jit, static_argnames=())
def _gather_agg(x_rows, adj_w, w_bcast):
    mesh = plsc.VectorSubcoreMesh(core_axis_name="c", subcore_axis_name="s")
    return pl.kernel(
        _gather_agg_body,
        out_type=jax.ShapeDtypeStruct((NPAD, ROWW), jnp.float32),
        mesh=mesh,
        scratch_types=[
            pltpu.VMEM((N_CHUNKS, CK), jnp.int32),
            pltpu.VMEM((CK, ROWW), jnp.float32),
            pltpu.VMEM((CK, ROWW), jnp.float32),
            pltpu.VMEM((CHUNK, ROWW), jnp.float32),
            pltpu.VMEM((CHUNK, ROWW), jnp.float32),
            pltpu.VMEM((K, ROWW), jnp.float32),
            pltpu.SemaphoreType.DMA,
            pltpu.SemaphoreType.DMA,
            pltpu.SemaphoreType.DMA,
            pltpu.SemaphoreType.DMA,
        ],
    )(x_rows, adj_w, w_bcast)


def _init_body(lat_ref, win_ref, bin_ref, pos_ref, out_ref):
    x0 = jnp.dot(lat_ref[...], win_ref[...],
                 preferred_element_type=jnp.float32) + bin_ref[...]
    out_ref[...] = pos_ref[...][:, None, :] + x0[None, :, :]


def _init_x(latent, W_in, b_in, pos_pad):
    tn = 1024
    return pl.pallas_call(
        _init_body,
        grid=(NPAD // tn,),
        in_specs=[
            pl.BlockSpec((B, W_in.shape[0]), lambda i: (0, 0)),
            pl.BlockSpec((W_in.shape[0], D), lambda i: (0, 0)),
            pl.BlockSpec((1, D), lambda i: (0, 0)),
            pl.BlockSpec((tn, D), lambda i: (i, 0)),
        ],
        out_specs=pl.BlockSpec((tn, B, D), lambda i: (i, 0, 0)),
        out_shape=jax.ShapeDtypeStruct((NPAD, B, D), jnp.float32),
    )(latent, W_in, b_in.reshape(1, D), pos_pad)


def _ln(x, g, b):
    m = jnp.mean(x, axis=-1, keepdims=True)
    v = jnp.mean((x - m) ** 2, axis=-1, keepdims=True)
    return (x - m) * lax.rsqrt(v + 1e-5) * g + b


def _block_body(x_ref, agg_ref, g_ref, b_ref, w1_ref, b1_ref, w2_ref, b2_ref,
                out_ref):
    h = _ln(agg_ref[...], g_ref[...], b_ref[...])
    u = jax.nn.gelu(jnp.dot(h, w1_ref[...], preferred_element_type=jnp.float32)
                    + b1_ref[...])
    y = jnp.dot(u, w2_ref[...], preferred_element_type=jnp.float32) + b2_ref[...]
    out_ref[...] = x_ref[...] + y


def _mlp_block(x2d, agg2d, g, b, W1, b1, W2, b2):
    r = 2048
    h4 = 4 * D
    return pl.pallas_call(
        _block_body,
        grid=(NROWS // r,),
        in_specs=[
            pl.BlockSpec((r, D), lambda i: (i, 0)),
            pl.BlockSpec((r, D), lambda i: (i, 0)),
            pl.BlockSpec((1, D), lambda i: (0, 0)),
            pl.BlockSpec((1, D), lambda i: (0, 0)),
            pl.BlockSpec((D, h4), lambda i: (0, 0)),
            pl.BlockSpec((1, h4), lambda i: (0, 0)),
            pl.BlockSpec((h4, D), lambda i: (0, 0)),
            pl.BlockSpec((1, D), lambda i: (0, 0)),
        ],
        out_specs=pl.BlockSpec((r, D), lambda i: (i, 0)),
        out_shape=jax.ShapeDtypeStruct((NROWS, D), jnp.float32),
    )(x2d, agg2d, g.reshape(1, D), b.reshape(1, D), W1, b1.reshape(1, h4),
      W2, b2.reshape(1, D))


def _head_body(x_ref, g_ref, b_ref, wh_ref, bh_ref, out_ref):
    h = _ln(x_ref[...], g_ref[...], b_ref[...])
    out_ref[...] = (jnp.dot(h, wh_ref[...], preferred_element_type=jnp.float32)
                    + bh_ref[...])


def _head(x2d, g, b, W_head, b_head):
    r = 2048
    return pl.pallas_call(
        _head_body,
        grid=(NROWS // r,),
        in_specs=[
            pl.BlockSpec((r, D), lambda i: (i, 0)),
            pl.BlockSpec((1, D), lambda i: (0, 0)),
            pl.BlockSpec((1, D), lambda i: (0, 0)),
            pl.BlockSpec((D, OUT), lambda i: (0, 0)),
            pl.BlockSpec((1, OUT), lambda i: (0, 0)),
        ],
        out_specs=pl.BlockSpec((r, OUT), lambda i: (i, 0)),
        out_shape=jax.ShapeDtypeStruct((NROWS, OUT), jnp.float32),
    )(x2d, g.reshape(1, D), b.reshape(1, D), W_head, b_head.reshape(1, OUT))


def kernel(latent_token, adj, W_in, b_in, pos_embed, w_nb, ln1_g, ln1_b,
           W1, b1, W2, b2, lnh_g, lnh_b, W_head, b_head):
    depth = w_nb.shape[0]
    pos_pad = jnp.zeros((NPAD, D), jnp.float32).at[:N].set(pos_embed[0])
    adj_flat = jnp.zeros((NPAD, K), jnp.int32).at[:N].set(
        adj.astype(jnp.int32)).reshape(NW, N_CHUNKS, CK)
    x = _init_x(latent_token, W_in, b_in, pos_pad)
    x = x.reshape(NPAD, ROWW)
    for i in range(depth):
        w_rows = jnp.tile(w_nb[i], (1, B)).astype(jnp.float32)
        agg = _gather_agg(x, adj_flat, w_rows)
        x = _mlp_block(x.reshape(NROWS, D), agg.reshape(NROWS, D),
                       ln1_g[i], ln1_b[i], W1[i], b1[i], W2[i], b2[i])
        x = x.reshape(NPAD, ROWW)
    y = _head(x.reshape(NROWS, D), lnh_g, lnh_b, W_head, b_head)
    out = y.reshape(NPAD, B, OUT)[:N]
    return jnp.transpose(out, (1, 2, 0))
